# baseline (speedup 1.0000x reference)
import jax
import jax.numpy as jnp
from jax.experimental import pallas as pl
from jax.experimental.pallas import tpu as pltpu

_B, _D, _A, _S = 16384, 128, 64, 64
_BT = 8192
_NROW = 192
_NPAD = _NROW - (_A + _S + 1)


def _tc_body(state_ref, w_ref, b_ref, mean_ref, std_ref, value_ref, onehot_ref):
    x = state_ref[...]
    w = w_ref[...]
    y = jax.lax.dot_general(w, x, (((1,), (1,)), ((), ())),
                            preferred_element_type=jnp.float32)
    y = y + b_ref[...]
    mean_ref[...] = y[:_A, :]
    std_ref[...] = jnp.zeros((_A, _BT), jnp.float32)
    logits = y[_A:_A + _S, :]
    m = jnp.max(logits, axis=0, keepdims=True)
    iota = jax.lax.broadcasted_iota(jnp.int32, (_S, _BT), 0)
    first = jnp.min(jnp.where(logits == m, iota, _S), axis=0, keepdims=True)
    onehot_ref[...] = (iota == first).astype(jnp.float32)
    value_ref[...] = y[_A + _S, :]


def kernel(state, W_skill, b_skill, W_action, b_action, W_value, b_value):
    w_rows = jnp.concatenate(
        [W_action, W_skill, W_value, jnp.zeros((_NPAD, _D), jnp.float32)],
        axis=0)
    b_col = jnp.concatenate(
        [b_action, b_skill, b_value,
         jnp.zeros((_NPAD,), jnp.float32)]).reshape(_NROW, 1)

    mean_t, std_t, value, onehot_t = pl.pallas_call(
        _tc_body,
        grid=(_B // _BT,),
        in_specs=[
            pl.BlockSpec((_BT, _D), lambda i: (i, 0)),
            pl.BlockSpec((_NROW, _D), lambda i: (0, 0)),
            pl.BlockSpec((_NROW, 1), lambda i: (0, 0)),
        ],
        out_specs=[
            pl.BlockSpec((_A, _BT), lambda i: (0, i)),
            pl.BlockSpec((_A, _BT), lambda i: (0, i)),
            pl.BlockSpec((_BT,), lambda i: (i,)),
            pl.BlockSpec((_S, _BT), lambda i: (0, i)),
        ],
        out_shape=[
            jax.ShapeDtypeStruct((_A, _B), jnp.float32),
            jax.ShapeDtypeStruct((_A, _B), jnp.float32),
            jax.ShapeDtypeStruct((_B,), jnp.float32),
            jax.ShapeDtypeStruct((_S, _B), jnp.float32),
        ],
        compiler_params=pltpu.CompilerParams(
            dimension_semantics=("arbitrary",),
        ),
    )(state, w_rows, b_col)

    return (mean_t.T, std_t.T, value, onehot_t.T)
